# Initial kernel scaffold; baseline (speedup 1.0000x reference)
import jax
import jax.numpy as jnp
from jax.experimental import pallas as pl


def kernel(x, edge_index, w1, att1_src, att1_dst, b1, w2, att2_src, att2_dst, b2, w3, b3):
    raise NotImplementedError("write your pallas kernel here")


# speedup vs baseline: 19.4041x; 19.4041x over previous
import functools

import jax
import jax.numpy as jnp
from jax import lax
from jax.experimental import pallas as pl
from jax.experimental.pallas import tpu as pltpu
from jax.experimental.pallas import tpu_sc as plsc

N = 10000
E = 320000
H1, C1 = 4, 64
H2, C2 = 4, 32
NCLS = 40

_CH = 80
_NSUB = 16
_DEN = 8
_NP = 10240
_RPT = _NP // _NSUB
_DPT = _RPT * _DEN


def _f32(x):
    return jnp.full((16,), x, dtype=jnp.float32)


def _i32(x):
    return jnp.full((16,), x, dtype=jnp.int32)


def _make_gat_sc(d):
    nw = E // _NSUB // _CH
    assert E % (_NSUB * _CH) == 0 and _NP % (_NSUB * 128) == 0
    mesh = plsc.VectorSubcoreMesh(core_axis_name="c", subcore_axis_name="s")

    @functools.partial(
        pl.kernel,
        out_type=(jax.ShapeDtypeStruct((2 * _NP, d), jnp.float32),
                  jax.ShapeDtypeStruct((2 * _NP * _DEN,), jnp.float32)),
        mesh=mesh,
        compiler_params=pltpu.CompilerParams(
            needs_layout_passes=False,
            use_tc_tiling_on_sc=(d == 128)),
        scratch_types=[
            pltpu.VMEM((16, d), jnp.float32),
            pltpu.VMEM((1024,), jnp.float32),
            pltpu.VMEM((_CH, d), jnp.float32),
            pltpu.VMEM((_CH,), jnp.int32),
            pltpu.VMEM((_CH,), jnp.int32),
            pltpu.VMEM((_CH,), jnp.int32),
            pltpu.VMEM((3, _CH), jnp.float32),
            pltpu.VMEM((3, _CH), jnp.int32),
            pltpu.VMEM((4, _CH), jnp.int32),
            pltpu.VMEM((4, _CH), jnp.float32),
            pltpu.VMEM((_CH, 1), jnp.int32),
            pltpu.VMEM((2, _CH), jnp.float32),
            pltpu.SemaphoreType.DMA,
            pltpu.VMEM_SHARED((_NP, d), jnp.float32),
            pltpu.VMEM_SHARED((_NP * _DEN,), jnp.float32),
            pltpu.VMEM_SHARED((2 * _NSUB, _CH), jnp.float32),
        ],
    )
    def k(src_h, dst_h, tab_h, hr_h, num_h, den_h,
          zb2, zbd, rowbuf, sraw, sadj, dbuf, vbuf, ibuf, tib, tvb, drow, vbuf2,
          ssem, acc, dens, vtmp):
        c = lax.axis_index("c")
        s = lax.axis_index("s")
        z16 = _f32(0.0)

        def zrow(r, _):
            for q in range(d // 16):
                zb2[r, pl.ds(q * 16, 16)] = z16
            return 0
        lax.fori_loop(0, 16, zrow, 0)

        def zw(i, _):
            zbd[pl.ds(i * 16, 16)] = z16
            return 0
        lax.fori_loop(0, 64, zw, 0)

        def zacc(i, _):
            pltpu.sync_copy(zb2, acc.at[pl.ds(s * _RPT + i * 16, 16)])
            return 0
        lax.fori_loop(0, _RPT // 16, zacc, 0)

        def zden(i, _):
            pltpu.sync_copy(zbd, dens.at[pl.ds(s * _DPT + i * 1024, 1024)])
            return 0
        lax.fori_loop(0, _DPT // 1024, zden, 0)

        for q in range(_CH // 16):
            vbuf[2, pl.ds(q * 16, 16)] = _f32(1.0)

        plsc.subcore_barrier()

        coff = lax.broadcast(c * N, (16,))
        ep_base = s * (E // _NSUB)

        tb = c * (N * 4)

        def window(w, _):
            base = ep_base + w * _CH
            pltpu.sync_copy(src_h.at[pl.ds(base, _CH)], sraw)
            pltpu.sync_copy(dst_h.at[pl.ds(base, _CH)], dbuf)
            for q in range(_CH // 16):
                sl = pl.ds(q * 16, 16)
                sv = sraw[sl]
                sadj[sl] = sv + coff
                dv = dbuf[sl]
                sv4 = sv * 4 + lax.broadcast(tb, (16,))
                dv4 = dv * 4 + lax.broadcast(tb, (16,))
                tib[0, sl] = sv4
                tib[1, sl] = sv4 + 1
                tib[2, sl] = dv4 + 2
                tib[3, sl] = dv4 + 3
                d8 = dv * _DEN
                ibuf[0, sl] = d8
                ibuf[1, sl] = d8 + 1
                ibuf[2, sl] = d8 + 2
                plsc.store_scatter(
                    drow, [lax.iota(jnp.int32, 16) + q * 16, _i32(0)], dv)
            for ch in range(4):
                pltpu.sync_copy(tab_h.at[tib.at[ch]], tvb.at[ch])
            pltpu.sync_copy(hr_h.at[sadj], rowbuf)
            exs = []
            for q in range(_CH // 16):
                sl = pl.ds(q * 16, 16)
                al0 = tvb[0, sl] + tvb[2, sl]
                al1 = tvb[1, sl] + tvb[3, sl]
                al0 = jnp.minimum(jnp.maximum(al0, 0.2 * al0), 60.0)
                al1 = jnp.minimum(jnp.maximum(al1, 0.2 * al1), 60.0)
                e0 = jnp.exp(al0)
                e1 = jnp.exp(al1)
                vbuf[0, sl] = e0
                vbuf[1, sl] = e1
                exs.append((e0, e1))
            for j in range(_CH):
                q, i = divmod(j, 16)
                oh = (lax.iota(jnp.int32, 16) == i).astype(jnp.float32)
                for hp in range(2):
                    cf = lax.broadcast(jnp.sum(exs[q][hp] * oh), (16,))
                    for qq in range(d // 32):
                        col = hp * (d // 2) + qq * 16
                        rowbuf[j, pl.ds(col, 16)] = rowbuf[j, pl.ds(col, 16)] * cf
            descs = [
                pltpu.async_copy(rowbuf.at[pl.ds(j, 1)], acc.at[drow.at[j]],
                                 ssem, add=True)
                for j in range(_CH)
            ]
            for ds_ in descs:
                ds_.wait()
            for ch in range(3):
                pltpu.sync_copy(vbuf.at[ch], dens.at[ibuf.at[ch]], add=True)
            return 0

        lax.fori_loop(0, nw, window, 0)

        plsc.subcore_barrier()
        pltpu.sync_copy(acc.at[pl.ds(s * _RPT, _RPT)],
                        num_h.at[pl.ds(c * _NP + s * _RPT, _RPT)])
        pltpu.sync_copy(dens.at[pl.ds(s * _DPT, _DPT)],
                        den_h.at[pl.ds(c * (_NP * _DEN) + s * _DPT, _DPT)])

    return k


def _make_gcn_sc(d):
    epw = E // (2 * _NSUB)
    nw = epw // _CH
    assert E % (2 * _NSUB * _CH) == 0
    mesh = plsc.VectorSubcoreMesh(core_axis_name="c", subcore_axis_name="s")

    @functools.partial(
        pl.kernel,
        out_type=jax.ShapeDtypeStruct((2 * _NP, d), jnp.float32),
        mesh=mesh,
        compiler_params=pltpu.CompilerParams(
            needs_layout_passes=False,
            use_tc_tiling_on_sc=(d == 128)),
        scratch_types=[
            pltpu.VMEM((N,), jnp.float32),
            pltpu.VMEM((128, d), jnp.float32),
            pltpu.VMEM((_CH, d), jnp.float32),
            pltpu.VMEM((_CH,), jnp.int32),
            pltpu.VMEM((_CH,), jnp.int32),
            pltpu.VMEM((_CH,), jnp.float32),
            pltpu.VMEM((_CH,), jnp.float32),
            pltpu.VMEM((_CH, 1), jnp.int32),
            pltpu.SemaphoreType.DMA,
            pltpu.VMEM_SHARED((_NP, d), jnp.float32),
            pltpu.VMEM_SHARED((_NSUB, _CH), jnp.float32),
        ],
    )
    def k(src_h, dst_h, dinv_h, hr_h, num_h,
          dtab, zb2, rowbuf, sbuf, dbuf, cbuf, cbuf2, drow, ssem, acc, vtmp):
        c = lax.axis_index("c")
        s = lax.axis_index("s")
        z16 = _f32(0.0)

        def zrow(r, _):
            for q in range(d // 16):
                zb2[r, pl.ds(q * 16, 16)] = z16
            return 0
        lax.fori_loop(0, 128, zrow, 0)
        for i in range(_RPT // 128):
            pltpu.sync_copy(zb2, acc.at[pl.ds(s * _RPT + i * 128, 128)])

        pltpu.sync_copy(dinv_h, dtab)
        plsc.subcore_barrier()

        ep_base = (c * _NSUB + s) * epw

        def window(w, _):
            base = ep_base + w * _CH
            pltpu.sync_copy(src_h.at[pl.ds(base, _CH)], sbuf)
            pltpu.sync_copy(dst_h.at[pl.ds(base, _CH)], dbuf)
            cfs = []
            for q in range(_CH // 16):
                sl = pl.ds(q * 16, 16)
                dv = dbuf[sl]
                dv_s = plsc.load_gather(dtab, [sbuf[sl]])
                dv_d = plsc.load_gather(dtab, [dv])
                cfs.append(dv_s * dv_d)
                plsc.store_scatter(
                    drow, [lax.iota(jnp.int32, 16) + q * 16, _i32(0)], dv)
            pltpu.sync_copy(hr_h.at[sbuf], rowbuf)
            for j in range(_CH):
                q, i = divmod(j, 16)
                oh = (lax.iota(jnp.int32, 16) == i).astype(jnp.float32)
                cf = lax.broadcast(jnp.sum(cfs[q] * oh), (16,))
                for qq in range(d // 16):
                    rowbuf[j, pl.ds(qq * 16, 16)] = rowbuf[j, pl.ds(qq * 16, 16)] * cf
            descs = [
                pltpu.async_copy(rowbuf.at[pl.ds(j, 1)], acc.at[drow.at[j]],
                                 ssem, add=True)
                for j in range(_CH)
            ]
            for ds_ in descs:
                ds_.wait()
            return 0

        lax.fori_loop(0, nw, window, 0)

        plsc.subcore_barrier()
        pltpu.sync_copy(acc.at[pl.ds(s * _RPT, _RPT)],
                        num_h.at[pl.ds(c * _NP + s * _RPT, _RPT)])

    return k


_BLK = 1000


def _proj_kernel(x_ref, w_ref, a_ref, h_ref, s_ref):
    h = jnp.dot(x_ref[...], w_ref[...], preferred_element_type=jnp.float32)
    h_ref[...] = h
    s_ref[...] = jnp.dot(h, a_ref[...], preferred_element_type=jnp.float32)


def _tc_proj(x, w, a):
    din, dout = w.shape
    return pl.pallas_call(
        _proj_kernel,
        grid=(N // _BLK,),
        in_specs=[
            pl.BlockSpec((_BLK, din), lambda i: (i, 0)),
            pl.BlockSpec((din, dout), lambda i: (0, 0)),
            pl.BlockSpec((dout, 8), lambda i: (0, 0)),
        ],
        out_specs=[
            pl.BlockSpec((_BLK, dout), lambda i: (i, 0)),
            pl.BlockSpec((_BLK, 8), lambda i: (i, 0)),
        ],
        out_shape=[
            jax.ShapeDtypeStruct((N, dout), jnp.float32),
            jax.ShapeDtypeStruct((N, 8), jnp.float32),
        ],
    )(x, w, a)


def _selfex(asad):
    cols = []
    for h in range(4):
        base = (h // 2) * 4
        a = asad[:, base + (h % 2):base + (h % 2) + 1] \
            + asad[:, base + 2 + (h % 2):base + 3 + (h % 2)]
        a = jnp.minimum(jnp.maximum(a, 0.2 * a), 60.0)
        cols.append(jnp.exp(a))
    return cols


def _mk_epilogue(c_width):
    def body(n0_ref, n1_ref, d0_ref, d1_ref, h_ref, s_ref, b_ref, w_ref,
             a_ref, h2_ref, s2_ref):
        exs = _selfex(s_ref[...])
        parts = []
        for h in range(4):
            nsc = (n0_ref if h < 2 else n1_ref)[...][:, (h % 2) * c_width:
                                                      (h % 2 + 1) * c_width]
            dsc = (d0_ref if h < 2 else d1_ref)[...][:, (h % 2):(h % 2) + 1]
            hblk = h_ref[...][:, h * c_width:(h + 1) * c_width]
            num = nsc + hblk * exs[h]
            den = dsc + exs[h]
            parts.append(num / den + b_ref[...][:, h * c_width:(h + 1) * c_width])
        x2 = jax.nn.relu(jnp.concatenate(parts, axis=1))
        h2 = jnp.dot(x2, w_ref[...], preferred_element_type=jnp.float32)
        h2_ref[...] = h2
        s2_ref[...] = jnp.dot(h2, a_ref[...], preferred_element_type=jnp.float32)

    def run(n0, n1, d0, d1, hprev, sprev, b, w, a):
        din, dnext = w.shape
        return pl.pallas_call(
            body,
            grid=(N // _BLK,),
            in_specs=[
                pl.BlockSpec((_BLK, din // 2), lambda i: (i, 0)),
                pl.BlockSpec((_BLK, din // 2), lambda i: (i, 0)),
                pl.BlockSpec((_BLK, _DEN), lambda i: (i, 0)),
                pl.BlockSpec((_BLK, _DEN), lambda i: (i, 0)),
                pl.BlockSpec((_BLK, din), lambda i: (i, 0)),
                pl.BlockSpec((_BLK, 8), lambda i: (i, 0)),
                pl.BlockSpec((1, din), lambda i: (0, 0)),
                pl.BlockSpec((din, dnext), lambda i: (0, 0)),
                pl.BlockSpec((dnext, 8), lambda i: (0, 0)),
            ],
            out_specs=[
                pl.BlockSpec((_BLK, dnext), lambda i: (i, 0)),
                pl.BlockSpec((_BLK, 8), lambda i: (i, 0)),
            ],
            out_shape=[
                jax.ShapeDtypeStruct((N, dnext), jnp.float32),
                jax.ShapeDtypeStruct((N, 8), jnp.float32),
            ],
        )(n0, n1, d0, d1, hprev, sprev, b, w, a)

    return run


def _gcn_prep_body(n0_ref, n1_ref, d0_ref, d1_ref, h_ref, s_ref, b_ref,
                   w_ref, dl1_ref, h3_ref, dinv_ref):
    exs = _selfex(s_ref[...])
    cw = 32
    parts = []
    for h in range(4):
        nsc = (n0_ref if h < 2 else n1_ref)[...][:, (h % 2) * cw:(h % 2 + 1) * cw]
        dsc = (d0_ref if h < 2 else d1_ref)[...][:, (h % 2):(h % 2) + 1]
        hblk = h_ref[...][:, h * cw:(h + 1) * cw]
        parts.append((nsc + hblk * exs[h]) / (dsc + exs[h])
                     + b_ref[...][:, h * cw:(h + 1) * cw])
    x3 = jax.nn.relu(jnp.concatenate(parts, axis=1))
    h3_ref[...] = jnp.dot(x3, w_ref[...], preferred_element_type=jnp.float32)
    deg = dl1_ref[...][:, 2:3] + 1.0
    dinv_ref[...] = lax.rsqrt(deg)


def _tc_gcn_prep(n0, n1, d0, d1, h2, s2, b2, w3p, den1_0):
    return pl.pallas_call(
        _gcn_prep_body,
        grid=(N // _BLK,),
        in_specs=[
            pl.BlockSpec((_BLK, 64), lambda i: (i, 0)),
            pl.BlockSpec((_BLK, 64), lambda i: (i, 0)),
            pl.BlockSpec((_BLK, _DEN), lambda i: (i, 0)),
            pl.BlockSpec((_BLK, _DEN), lambda i: (i, 0)),
            pl.BlockSpec((_BLK, 128), lambda i: (i, 0)),
            pl.BlockSpec((_BLK, 8), lambda i: (i, 0)),
            pl.BlockSpec((1, 128), lambda i: (0, 0)),
            pl.BlockSpec((128, 64), lambda i: (0, 0)),
            pl.BlockSpec((_BLK, _DEN), lambda i: (i, 0)),
        ],
        out_specs=[
            pl.BlockSpec((_BLK, 64), lambda i: (i, 0)),
            pl.BlockSpec((_BLK, 1), lambda i: (i, 0)),
        ],
        out_shape=[
            jax.ShapeDtypeStruct((N, 64), jnp.float32),
            jax.ShapeDtypeStruct((N, 1), jnp.float32),
        ],
    )(n0, n1, d0, d1, h2, s2, b2, w3p, den1_0)


def _final_body(n0_ref, n1_ref, h3_ref, dinv_ref, b_ref, o_ref):
    dinv = dinv_ref[...]
    o = n0_ref[...] + n1_ref[...] + h3_ref[...] * (dinv * dinv) + b_ref[...]
    mask = lax.broadcasted_iota(jnp.int32, o.shape, 1) < NCLS
    om = jnp.where(mask, o, -1e30)
    m = jnp.max(om, axis=1, keepdims=True)
    z = jnp.where(mask, jnp.exp(om - m), 0.0)
    o_ref[...] = om - m - jnp.log(jnp.sum(z, axis=1, keepdims=True))


def _tc_final(n0, n1, h3, dinv, b3p):
    return pl.pallas_call(
        _final_body,
        grid=(N // _BLK,),
        in_specs=[
            pl.BlockSpec((_BLK, 64), lambda i: (i, 0)),
            pl.BlockSpec((_BLK, 64), lambda i: (i, 0)),
            pl.BlockSpec((_BLK, 64), lambda i: (i, 0)),
            pl.BlockSpec((_BLK, 1), lambda i: (i, 0)),
            pl.BlockSpec((1, 64), lambda i: (0, 0)),
        ],
        out_specs=pl.BlockSpec((_BLK, 64), lambda i: (i, 0)),
        out_shape=jax.ShapeDtypeStruct((N, 64), jnp.float32),
    )(n0, n1, h3, dinv, b3p)


def _att_mat(att_src, att_dst, heads, ch):
    m = jnp.kron(jnp.eye(heads, dtype=jnp.float32),
                 jnp.ones((ch, 1), dtype=jnp.float32))
    s = m * att_src.reshape(-1, 1)
    d = m * att_dst.reshape(-1, 1)
    return jnp.concatenate([s[:, :2], d[:, :2], s[:, 2:], d[:, 2:]], axis=1)


_gat_sc_128 = _make_gat_sc(128)
_gat_sc_64 = _make_gat_sc(64)
_gcn_sc_64 = _make_gcn_sc(64)
_epi1 = _mk_epilogue(64)


def kernel(x, edge_index, w1, att1_src, att1_dst, b1,
           w2, att2_src, att2_dst, b2, w3, b3):
    src = edge_index[0]
    dst = edge_index[1]

    a1 = _att_mat(att1_src, att1_dst, H1, C1)
    a2 = _att_mat(att2_src, att2_dst, H2, C2)
    w3p = jnp.zeros((H2 * C2, 64), jnp.float32).at[:, :NCLS].set(w3)
    b3p = jnp.zeros((1, 64), jnp.float32).at[0, :NCLS].set(b3)

    h1, asad1 = _tc_proj(x, w1, a1)
    h1_2n = h1.reshape(N, 2, 128).transpose(1, 0, 2).reshape(2 * N, 128)
    tab1 = asad1.reshape(N, 2, 4).transpose(1, 0, 2).reshape(2 * N * 4)
    num1, den1 = _gat_sc_128(src, dst, tab1, h1_2n)
    den1 = den1.reshape(2, _NP, _DEN)[:, :N]

    h2, asad2 = _epi1(num1[:N], num1[_NP:_NP + N], den1[0], den1[1],
                      h1, asad1, b1.reshape(1, -1), w2, a2)
    h2_2n = h2.reshape(N, 2, 64).transpose(1, 0, 2).reshape(2 * N, 64)
    tab2 = asad2.reshape(N, 2, 4).transpose(1, 0, 2).reshape(2 * N * 4)
    num2, den2 = _gat_sc_64(src, dst, tab2, h2_2n)
    den2 = den2.reshape(2, _NP, _DEN)[:, :N]

    h3, dinv = _tc_gcn_prep(num2[:N], num2[_NP:_NP + N], den2[0], den2[1],
                            h2, asad2, b2.reshape(1, -1), w3p, den1[0])
    num3 = _gcn_sc_64(src, dst, dinv.reshape(N), h3)

    out = _tc_final(num3[:N], num3[_NP:_NP + N], h3, dinv, b3p)
    return out[:, :NCLS]
